# baseline (speedup 1.0000x reference)
import functools

import jax
import jax.numpy as jnp
from jax import lax
from jax.experimental import pallas as pl
from jax.experimental.pallas import tpu as pltpu
from jax.experimental.pallas import tpu_sc as plsc

V = 100000
D = 64
K = 8
B = 16384

NC = 2
NS = 16
NW = NC * NS
L = 16

BPW = B // NW
CHUNK = 128
NCHUNK = BPW // CHUNK
GROUPS = CHUNK // L


def _splat(val, dtype=jnp.int32):
    return jnp.full((L,), val, dtype=dtype)


def _sense_kernel(word_hbm, ctxh_hbm, ctxo_hbm, tgth_hbm, tgto_hbm,
                  wg_hbm, ws_hbm, out_hbm,
                  word_v, ctxh_v, ctxo_v, tgth_v, tgto_v,
                  sense_v, ctxr_v, tgtr_v, out_v, sem):
    wid = lax.axis_index("s") * NC + lax.axis_index("c")

    for chunk in range(NCHUNK):
        base = wid * BPW + chunk * CHUNK

        pltpu.sync_copy(word_hbm.at[pl.ds(base, CHUNK)], word_v)
        pltpu.sync_copy(ctxh_hbm.at[pl.ds(base, CHUNK)], ctxh_v)
        pltpu.sync_copy(ctxo_hbm.at[pl.ds(base, CHUNK)], ctxo_v)
        pltpu.sync_copy(tgth_hbm.at[pl.ds(base, CHUNK)], tgth_v)
        pltpu.sync_copy(tgto_hbm.at[pl.ds(base, CHUNK)], tgto_v)

        c1 = pltpu.async_copy(ws_hbm.at[word_v], sense_v, sem)
        c2 = pltpu.async_copy(wg_hbm.at[ctxh_v], ctxr_v, sem)
        c3 = pltpu.async_copy(wg_hbm.at[tgth_v], tgtr_v, sem)
        c1.wait()
        c2.wait()
        c3.wait()

        def group_body(g, _):
            tok = g * L + lax.iota(jnp.int32, L)
            ctxoff = ctxo_v[pl.ds(g * L, L)]
            tgtoff = tgto_v[pl.ds(g * L, L)]

            def score_body(d, accs):
                ctxv = plsc.load_gather(ctxr_v, [tok, ctxoff + d])
                d8 = d * K
                new = []
                for k in range(K):
                    sv = plsc.load_gather(sense_v, [tok, _splat(d8 + k)])
                    new.append(accs[k] + ctxv * sv)
                return tuple(new)

            zeros = _splat(0.0, jnp.float32)
            accs = lax.fori_loop(0, D, score_body, (zeros,) * K)

            best = accs[0]
            bidx = _splat(0)
            for k in range(1, K):
                m = accs[k] > best
                best = jnp.where(m, accs[k], best)
                bidx = jnp.where(m, _splat(k), bidx)

            def dot_body(d, acc):
                chosen = plsc.load_gather(sense_v, [tok, _splat(d * K) + bidx])
                tv = plsc.load_gather(tgtr_v, [tok, tgtoff + d])
                return acc + chosen * tv

            dot = lax.fori_loop(0, D, dot_body, zeros)
            res = 1.0 / (1.0 + jnp.exp(-dot))
            out_v[pl.ds(g * L, L)] = res
            return 0

        lax.fori_loop(0, GROUPS, group_body, 0)
        pltpu.sync_copy(out_v, out_hbm.at[pl.ds(base, CHUNK)])


@jax.jit
def _run(word, ctx_hi, ctx_off, tgt_hi, tgt_off, wg2, ws2):
    mesh = plsc.VectorSubcoreMesh(core_axis_name="c", subcore_axis_name="s")
    f = functools.partial(
        pl.kernel,
        mesh=mesh,
        compiler_params=pltpu.CompilerParams(needs_layout_passes=False),
        out_type=jax.ShapeDtypeStruct((B,), jnp.float32),
        scratch_types=[
            pltpu.VMEM((CHUNK,), jnp.int32),
            pltpu.VMEM((CHUNK,), jnp.int32),
            pltpu.VMEM((CHUNK,), jnp.int32),
            pltpu.VMEM((CHUNK,), jnp.int32),
            pltpu.VMEM((CHUNK,), jnp.int32),
            pltpu.VMEM((CHUNK, D * K), jnp.float32),
            pltpu.VMEM((CHUNK, 2 * D), jnp.float32),
            pltpu.VMEM((CHUNK, 2 * D), jnp.float32),
            pltpu.VMEM((CHUNK,), jnp.float32),
            pltpu.SemaphoreType.DMA,
        ],
    )(_sense_kernel)
    return f(word, ctx_hi, ctx_off, tgt_hi, tgt_off, wg2, ws2)


def kernel(x, W_g, W_s):
    word = x[0].astype(jnp.int32)
    ctx = x[1].astype(jnp.int32)
    tgt = x[2].astype(jnp.int32)
    ctx_hi = ctx >> 1
    ctx_off = (ctx & 1) * D
    tgt_hi = tgt >> 1
    tgt_off = (tgt & 1) * D
    wg2 = W_g.reshape(V // 2, 2 * D)
    ws2 = W_s.reshape(V, D * K)
    return _run(word, ctx_hi, ctx_off, tgt_hi, tgt_off, wg2, ws2)
